# baseline (speedup 1.0000x reference)
import jax
import jax.numpy as jnp
from jax import lax
from jax.experimental import pallas as pl
from jax.experimental.pallas import tpu as pltpu
from jax.experimental.pallas import tpu_sc as plsc

T = 4096
DIM = 1024
DFF = 2048
E = 8
K = 2

TILE_A = 512
TILE_F = 512
NTILES = (T * K) // TILE_F + E
NPAD = NTILES * TILE_F
DFFB = 2048
NDFF = DFF // DFFB
TE_LEN = 80

CTR = 256
NCHR = T // CTR

NC = 2
NS = 16
NW = NC * NS
TPW = T // NW
CH = 32
NCH = TPW // CH

_MESH = plsc.VectorSubcoreMesh(core_axis_name="c", subcore_axis_name="s",
                               num_cores=NC, num_subcores=NS)


def _route_body(x_ref, gw_ref, ti0_ref, ti1_ref, tw_ref):
    xb = x_ref[...]
    logits = lax.dot_general(xb, gw_ref[...], (((1,), (1,)), ((), ())),
                             preferred_element_type=jnp.float32)
    eids = lax.broadcasted_iota(jnp.int32, (1, E), 1)
    v1 = jnp.max(logits, axis=1, keepdims=True)
    i1 = jnp.argmax(logits, axis=1, keepdims=True).astype(jnp.int32)
    masked = jnp.where(eids == i1, -jnp.inf, logits)
    v2 = jnp.max(masked, axis=1, keepdims=True)
    i2 = jnp.argmax(masked, axis=1, keepdims=True).astype(jnp.int32)
    ti0_ref[...] = i1
    ti1_ref[...] = i2
    e2 = jnp.exp(v2 - v1)
    denom = 1.0 + e2
    tw_ref[...] = jnp.concatenate([1.0 / denom, e2 / denom], axis=1)


def _rank_body(ti0_ref, ti1_ref, pe_ref, po_ref, te_ref, carry_s, tot_s):
    p_idx = pl.program_id(0)
    c_idx = pl.program_id(1)

    @pl.when(jnp.logical_and(p_idx == 0, c_idx == 0))
    def _():
        carry_s[...] = jnp.zeros_like(carry_s)

    eids = lax.broadcasted_iota(jnp.int32, (1, E), 1)
    oh0 = jnp.where(ti0_ref[...] == eids, 1.0, 0.0)
    oh1 = jnp.where(ti1_ref[...] == eids, 1.0, 0.0)
    ohi = jnp.concatenate(
        [oh0.reshape(CTR, 1, E), oh1.reshape(CTR, 1, E)], axis=1
    ).reshape(2 * CTR, E)

    @pl.when(p_idx == 0)
    def _():
        carry_s[...] += jnp.sum(ohi, axis=0, keepdims=True)

    @pl.when(p_idx == 1)
    def _():
        @pl.when(c_idx == 0)
        def _():
            tot_s[...] = carry_s[...]
            carry_s[...] = jnp.zeros_like(carry_s)

        tot = tot_s[...]
        padded = jnp.floor((tot + (TILE_F - 1)) * (1.0 / TILE_F)) * TILE_F
        r8 = lax.broadcasted_iota(jnp.int32, (E, E), 0)
        c8 = lax.broadcasted_iota(jnp.int32, (E, E), 1)
        u8 = jnp.where(r8 < c8, 1.0, 0.0)
        pad_base = lax.dot_general(padded, u8, (((1,), (0,)), ((), ())),
                                   preferred_element_type=jnp.float32)
        base_vec = pad_base + carry_s[...]

        rr = lax.broadcasted_iota(jnp.int32, (2 * CTR, 2 * CTR), 0)
        cc = lax.broadcasted_iota(jnp.int32, (2 * CTR, 2 * CTR), 1)
        lt = jnp.where(cc < rr, 1.0, 0.0)
        within = lax.dot_general(lt, ohi, (((1,), (0,)), ((), ())),
                                 preferred_element_type=jnp.float32)
        basep = jnp.sum(ohi * base_vec, axis=1, keepdims=True)
        withinp = jnp.sum(within * ohi, axis=1, keepdims=True)
        pos = (basep + withinp).astype(jnp.int32).reshape(CTR, K)
        pe_ref[...] = pos[:, 0:1]
        po_ref[...] = pos[:, 1:2]
        carry_s[...] += jnp.sum(ohi, axis=0, keepdims=True)

        gi = lax.broadcasted_iota(jnp.int32, (TE_LEN, E), 0)
        ge = jnp.where((gi * TILE_F).astype(jnp.float32) >= pad_base,
                       1.0, 0.0)
        te_ref[...] = (jnp.sum(ge, axis=1, keepdims=True)
                       - 1.0).astype(jnp.int32)


def _sc_dispatch_body(x_hbm, pe_hbm, po_hbm, xs_hbm, pe2_v, po2_v, xbuf_v):
    wid = lax.axis_index("s") * NC + lax.axis_index("c")
    base = wid * TPW
    for ch in range(NCH):
        pltpu.sync_copy(pe_hbm.at[pl.ds(base + ch * CH, CH)], pe2_v.at[ch])
        pltpu.sync_copy(po_hbm.at[pl.ds(base + ch * CH, CH)], po2_v.at[ch])
    for ch in range(NCH):
        pltpu.sync_copy(x_hbm.at[pl.ds(base + ch * CH, CH)], xbuf_v)
        pltpu.sync_copy(xbuf_v, xs_hbm.at[pe2_v.at[ch]])
        pltpu.sync_copy(xbuf_v, xs_hbm.at[po2_v.at[ch]])


def _ffn_body(te_ref, x_ref, w1_ref, w3_ref, w2_ref, out_ref):
    f_idx = pl.program_id(1)

    @pl.when(f_idx == 0)
    def _():
        out_ref[...] = jnp.zeros_like(out_ref)

    xb = x_ref[...]
    h1 = lax.dot_general(xb, w1_ref[0], (((1,), (1,)), ((), ())),
                         preferred_element_type=jnp.float32)
    h3 = lax.dot_general(xb, w3_ref[0], (((1,), (1,)), ((), ())),
                         preferred_element_type=jnp.float32)
    hg = h1 * lax.logistic(h1) * h3
    out_ref[...] += lax.dot_general(hg, w2_ref[0], (((1,), (0,)), ((), ())),
                                    preferred_element_type=jnp.float32)


def _sc_unsort_body(os_hbm, pe_hbm, po_hbm, outa_hbm, outb_hbm,
                    idx2_v, gbuf_v):
    wid = lax.axis_index("s") * NC + lax.axis_index("c")
    base = wid * TPW
    for p_hbm, o_hbm in ((pe_hbm, outa_hbm), (po_hbm, outb_hbm)):
        for ch in range(NCH):
            t0 = base + ch * CH
            pltpu.sync_copy(p_hbm.at[pl.ds(t0, CH)], idx2_v.at[ch])
            pltpu.sync_copy(os_hbm.at[idx2_v.at[ch]], gbuf_v)
            pltpu.sync_copy(gbuf_v, o_hbm.at[pl.ds(t0, CH)])


def _combine_body(nr_ref, a_ref, b_ref, w_ref, out_ref):
    out_ref[...] = nr_ref[...] + w_ref[...] * (a_ref[...] + b_ref[...])


_sc_dispatch = pl.kernel(
    _sc_dispatch_body,
    out_type=jax.ShapeDtypeStruct((NPAD, DIM), jnp.float32),
    mesh=_MESH,
    scratch_types=[pltpu.VMEM((NCH, CH), jnp.int32),
                   pltpu.VMEM((NCH, CH), jnp.int32),
                   pltpu.VMEM((CH, DIM), jnp.float32)])

_sc_unsort = pl.kernel(
    _sc_unsort_body,
    out_type=[jax.ShapeDtypeStruct((T, DIM), jnp.float32),
              jax.ShapeDtypeStruct((T, DIM), jnp.float32)],
    mesh=_MESH,
    scratch_types=[pltpu.VMEM((NCH, CH), jnp.int32),
                   pltpu.VMEM((CH, DIM), jnp.float32)])


@jax.jit
def kernel(x, next_r, gate_w, w1, w2, w3):
    ti0, ti1, tw = pl.pallas_call(
        _route_body,
        grid=(T // TILE_A,),
        in_specs=[
            pl.BlockSpec((TILE_A, DIM), lambda g: (g, 0)),
            pl.BlockSpec((E, DIM), lambda g: (0, 0)),
        ],
        out_specs=[
            pl.BlockSpec((TILE_A, 1), lambda g: (g, 0)),
            pl.BlockSpec((TILE_A, 1), lambda g: (g, 0)),
            pl.BlockSpec((TILE_A, K), lambda g: (g, 0)),
        ],
        out_shape=[
            jax.ShapeDtypeStruct((T, 1), jnp.int32),
            jax.ShapeDtypeStruct((T, 1), jnp.int32),
            jax.ShapeDtypeStruct((T, K), jnp.float32),
        ],
    )(x, gate_w)

    w_used = tw[: T // K].reshape(T, 1)

    pe2, po2, te2 = pl.pallas_call(
        _rank_body,
        grid=(2, NCHR),
        in_specs=[
            pl.BlockSpec((CTR, 1), lambda p, c: (c, 0)),
            pl.BlockSpec((CTR, 1), lambda p, c: (c, 0)),
        ],
        out_specs=[
            pl.BlockSpec((CTR, 1), lambda p, c: (c, 0)),
            pl.BlockSpec((CTR, 1), lambda p, c: (c, 0)),
            pl.BlockSpec((TE_LEN, 1), lambda p, c: (0, 0)),
        ],
        out_shape=[
            jax.ShapeDtypeStruct((T, 1), jnp.int32),
            jax.ShapeDtypeStruct((T, 1), jnp.int32),
            jax.ShapeDtypeStruct((TE_LEN, 1), jnp.int32),
        ],
        scratch_shapes=[
            pltpu.VMEM((1, E), jnp.float32),
            pltpu.VMEM((1, E), jnp.float32),
        ],
    )(ti0, ti1)

    pe = pe2.reshape(T)
    po = po2.reshape(T)
    te = te2.reshape(TE_LEN)

    xs = _sc_dispatch(x, pe, po)

    outs = pl.pallas_call(
        _ffn_body,
        grid_spec=pltpu.PrefetchScalarGridSpec(
            num_scalar_prefetch=1,
            grid=(NTILES, NDFF),
            in_specs=[
                pl.BlockSpec((TILE_F, DIM), lambda g, f, te_r: (g, 0)),
                pl.BlockSpec((1, DFFB, DIM),
                             lambda g, f, te_r: (te_r[g], f, 0)),
                pl.BlockSpec((1, DFFB, DIM),
                             lambda g, f, te_r: (te_r[g], f, 0)),
                pl.BlockSpec((1, DFFB, DIM),
                             lambda g, f, te_r: (te_r[g], f, 0)),
            ],
            out_specs=pl.BlockSpec((TILE_F, DIM), lambda g, f, te_r: (g, 0)),
        ),
        out_shape=jax.ShapeDtypeStruct((NPAD, DIM), jnp.float32),
        compiler_params=pltpu.CompilerParams(
            vmem_limit_bytes=63 * 1024 * 1024),
    )(te, xs, w1, w3, w2)

    outa, outb = _sc_unsort(outs, pe, po)

    out = pl.pallas_call(
        _combine_body,
        grid=(4,),
        in_specs=[
            pl.BlockSpec((T // 4, DIM), lambda g: (g, 0)),
            pl.BlockSpec((T // 4, DIM), lambda g: (g, 0)),
            pl.BlockSpec((T // 4, DIM), lambda g: (g, 0)),
            pl.BlockSpec((T // 4, 1), lambda g: (g, 0)),
        ],
        out_specs=pl.BlockSpec((T // 4, DIM), lambda g: (g, 0)),
        out_shape=jax.ShapeDtypeStruct((T, DIM), jnp.float32),
    )(next_r, outa, outb, w_used)
    return out


# speedup vs baseline: 1.0238x; 1.0238x over previous
import jax
import jax.numpy as jnp
from jax import lax
from jax.experimental import pallas as pl
from jax.experimental.pallas import tpu as pltpu
from jax.experimental.pallas import tpu_sc as plsc

T = 4096
DIM = 1024
DFF = 2048
E = 8
K = 2

TILE_A = 512
TILE_F = 256
NTILES = (T * K) // TILE_F + E
NPAD = NTILES * TILE_F
DFFB = 2048
NDFF = DFF // DFFB
TE_LEN = 80

CTR = 256
NCHR = T // CTR

NC = 2
NS = 16
NW = NC * NS
TPW = T // NW
CH = 32
NCH = TPW // CH

_MESH = plsc.VectorSubcoreMesh(core_axis_name="c", subcore_axis_name="s",
                               num_cores=NC, num_subcores=NS)


def _route_body(x_ref, gw_ref, ti0_ref, ti1_ref, tw_ref):
    xb = x_ref[...]
    logits = lax.dot_general(xb, gw_ref[...], (((1,), (1,)), ((), ())),
                             preferred_element_type=jnp.float32)
    eids = lax.broadcasted_iota(jnp.int32, (1, E), 1)
    v1 = jnp.max(logits, axis=1, keepdims=True)
    i1 = jnp.argmax(logits, axis=1, keepdims=True).astype(jnp.int32)
    masked = jnp.where(eids == i1, -jnp.inf, logits)
    v2 = jnp.max(masked, axis=1, keepdims=True)
    i2 = jnp.argmax(masked, axis=1, keepdims=True).astype(jnp.int32)
    ti0_ref[...] = i1
    ti1_ref[...] = i2
    e2 = jnp.exp(v2 - v1)
    denom = 1.0 + e2
    tw_ref[...] = jnp.concatenate([1.0 / denom, e2 / denom], axis=1)


def _rank_body(ti0_ref, ti1_ref, pe_ref, po_ref, te_ref, carry_s, tot_s):
    p_idx = pl.program_id(0)
    c_idx = pl.program_id(1)

    @pl.when(jnp.logical_and(p_idx == 0, c_idx == 0))
    def _():
        carry_s[...] = jnp.zeros_like(carry_s)

    eids = lax.broadcasted_iota(jnp.int32, (1, E), 1)
    oh0 = jnp.where(ti0_ref[...] == eids, 1.0, 0.0)
    oh1 = jnp.where(ti1_ref[...] == eids, 1.0, 0.0)
    ohi = jnp.concatenate(
        [oh0.reshape(CTR, 1, E), oh1.reshape(CTR, 1, E)], axis=1
    ).reshape(2 * CTR, E)

    @pl.when(p_idx == 0)
    def _():
        carry_s[...] += jnp.sum(ohi, axis=0, keepdims=True)

    @pl.when(p_idx == 1)
    def _():
        @pl.when(c_idx == 0)
        def _():
            tot_s[...] = carry_s[...]
            carry_s[...] = jnp.zeros_like(carry_s)

        tot = tot_s[...]
        padded = jnp.floor((tot + (TILE_F - 1)) * (1.0 / TILE_F)) * TILE_F
        r8 = lax.broadcasted_iota(jnp.int32, (E, E), 0)
        c8 = lax.broadcasted_iota(jnp.int32, (E, E), 1)
        u8 = jnp.where(r8 < c8, 1.0, 0.0)
        pad_base = lax.dot_general(padded, u8, (((1,), (0,)), ((), ())),
                                   preferred_element_type=jnp.float32)
        base_vec = pad_base + carry_s[...]

        rr = lax.broadcasted_iota(jnp.int32, (2 * CTR, 2 * CTR), 0)
        cc = lax.broadcasted_iota(jnp.int32, (2 * CTR, 2 * CTR), 1)
        lt = jnp.where(cc < rr, 1.0, 0.0)
        within = lax.dot_general(lt, ohi, (((1,), (0,)), ((), ())),
                                 preferred_element_type=jnp.float32)
        basep = jnp.sum(ohi * base_vec, axis=1, keepdims=True)
        withinp = jnp.sum(within * ohi, axis=1, keepdims=True)
        pos = (basep + withinp).astype(jnp.int32).reshape(CTR, K)
        pe_ref[...] = pos[:, 0:1]
        po_ref[...] = pos[:, 1:2]
        carry_s[...] += jnp.sum(ohi, axis=0, keepdims=True)

        gi = lax.broadcasted_iota(jnp.int32, (TE_LEN, E), 0)
        ge = jnp.where((gi * TILE_F).astype(jnp.float32) >= pad_base,
                       1.0, 0.0)
        te_ref[...] = (jnp.sum(ge, axis=1, keepdims=True)
                       - 1.0).astype(jnp.int32)


def _sc_dispatch_body(x_hbm, pe_hbm, po_hbm, xs_hbm, pe2_v, po2_v, xbuf_v):
    wid = lax.axis_index("s") * NC + lax.axis_index("c")
    base = wid * TPW
    for ch in range(NCH):
        pltpu.sync_copy(pe_hbm.at[pl.ds(base + ch * CH, CH)], pe2_v.at[ch])
        pltpu.sync_copy(po_hbm.at[pl.ds(base + ch * CH, CH)], po2_v.at[ch])
    for ch in range(NCH):
        pltpu.sync_copy(x_hbm.at[pl.ds(base + ch * CH, CH)], xbuf_v)
        pltpu.sync_copy(xbuf_v, xs_hbm.at[pe2_v.at[ch]])
        pltpu.sync_copy(xbuf_v, xs_hbm.at[po2_v.at[ch]])


def _ffn_body(te_ref, x_ref, w1_ref, w3_ref, w2_ref, out_ref):
    f_idx = pl.program_id(1)

    @pl.when(f_idx == 0)
    def _():
        out_ref[...] = jnp.zeros_like(out_ref)

    xb = x_ref[...]
    h1 = lax.dot_general(xb, w1_ref[0], (((1,), (1,)), ((), ())),
                         preferred_element_type=jnp.float32)
    h3 = lax.dot_general(xb, w3_ref[0], (((1,), (1,)), ((), ())),
                         preferred_element_type=jnp.float32)
    hg = h1 * lax.logistic(h1) * h3
    out_ref[...] += lax.dot_general(hg, w2_ref[0], (((1,), (0,)), ((), ())),
                                    preferred_element_type=jnp.float32)


def _sc_unsort_body(os_hbm, pe_hbm, po_hbm, outa_hbm, outb_hbm,
                    idx2_v, gbuf_v):
    wid = lax.axis_index("s") * NC + lax.axis_index("c")
    base = wid * TPW
    for p_hbm, o_hbm in ((pe_hbm, outa_hbm), (po_hbm, outb_hbm)):
        for ch in range(NCH):
            t0 = base + ch * CH
            pltpu.sync_copy(p_hbm.at[pl.ds(t0, CH)], idx2_v.at[ch])
            pltpu.sync_copy(os_hbm.at[idx2_v.at[ch]], gbuf_v)
            pltpu.sync_copy(gbuf_v, o_hbm.at[pl.ds(t0, CH)])


def _combine_body(nr_ref, a_ref, b_ref, w_ref, out_ref):
    out_ref[...] = nr_ref[...] + w_ref[...] * (a_ref[...] + b_ref[...])


_sc_dispatch = pl.kernel(
    _sc_dispatch_body,
    out_type=jax.ShapeDtypeStruct((NPAD, DIM), jnp.float32),
    mesh=_MESH,
    scratch_types=[pltpu.VMEM((NCH, CH), jnp.int32),
                   pltpu.VMEM((NCH, CH), jnp.int32),
                   pltpu.VMEM((CH, DIM), jnp.float32)])

_sc_unsort = pl.kernel(
    _sc_unsort_body,
    out_type=[jax.ShapeDtypeStruct((T, DIM), jnp.float32),
              jax.ShapeDtypeStruct((T, DIM), jnp.float32)],
    mesh=_MESH,
    scratch_types=[pltpu.VMEM((NCH, CH), jnp.int32),
                   pltpu.VMEM((CH, DIM), jnp.float32)])


@jax.jit
def kernel(x, next_r, gate_w, w1, w2, w3):
    ti0, ti1, tw = pl.pallas_call(
        _route_body,
        grid=(T // TILE_A,),
        in_specs=[
            pl.BlockSpec((TILE_A, DIM), lambda g: (g, 0)),
            pl.BlockSpec((E, DIM), lambda g: (0, 0)),
        ],
        out_specs=[
            pl.BlockSpec((TILE_A, 1), lambda g: (g, 0)),
            pl.BlockSpec((TILE_A, 1), lambda g: (g, 0)),
            pl.BlockSpec((TILE_A, K), lambda g: (g, 0)),
        ],
        out_shape=[
            jax.ShapeDtypeStruct((T, 1), jnp.int32),
            jax.ShapeDtypeStruct((T, 1), jnp.int32),
            jax.ShapeDtypeStruct((T, K), jnp.float32),
        ],
    )(x, gate_w)

    w_used = tw[: T // K].reshape(T, 1)

    pe2, po2, te2 = pl.pallas_call(
        _rank_body,
        grid=(2, NCHR),
        in_specs=[
            pl.BlockSpec((CTR, 1), lambda p, c: (c, 0)),
            pl.BlockSpec((CTR, 1), lambda p, c: (c, 0)),
        ],
        out_specs=[
            pl.BlockSpec((CTR, 1), lambda p, c: (c, 0)),
            pl.BlockSpec((CTR, 1), lambda p, c: (c, 0)),
            pl.BlockSpec((TE_LEN, 1), lambda p, c: (0, 0)),
        ],
        out_shape=[
            jax.ShapeDtypeStruct((T, 1), jnp.int32),
            jax.ShapeDtypeStruct((T, 1), jnp.int32),
            jax.ShapeDtypeStruct((TE_LEN, 1), jnp.int32),
        ],
        scratch_shapes=[
            pltpu.VMEM((1, E), jnp.float32),
            pltpu.VMEM((1, E), jnp.float32),
        ],
    )(ti0, ti1)

    pe = pe2.reshape(T)
    po = po2.reshape(T)
    te = te2.reshape(TE_LEN)

    xs = _sc_dispatch(x, pe, po)

    outs = pl.pallas_call(
        _ffn_body,
        grid_spec=pltpu.PrefetchScalarGridSpec(
            num_scalar_prefetch=1,
            grid=(NTILES, NDFF),
            in_specs=[
                pl.BlockSpec((TILE_F, DIM), lambda g, f, te_r: (g, 0)),
                pl.BlockSpec((1, DFFB, DIM),
                             lambda g, f, te_r: (te_r[g], f, 0)),
                pl.BlockSpec((1, DFFB, DIM),
                             lambda g, f, te_r: (te_r[g], f, 0)),
                pl.BlockSpec((1, DFFB, DIM),
                             lambda g, f, te_r: (te_r[g], f, 0)),
            ],
            out_specs=pl.BlockSpec((TILE_F, DIM), lambda g, f, te_r: (g, 0)),
        ),
        out_shape=jax.ShapeDtypeStruct((NPAD, DIM), jnp.float32),
        compiler_params=pltpu.CompilerParams(
            vmem_limit_bytes=63 * 1024 * 1024),
    )(te, xs, w1, w3, w2)

    outa, outb = _sc_unsort(outs, pe, po)

    out = pl.pallas_call(
        _combine_body,
        grid=(4,),
        in_specs=[
            pl.BlockSpec((T // 4, DIM), lambda g: (g, 0)),
            pl.BlockSpec((T // 4, DIM), lambda g: (g, 0)),
            pl.BlockSpec((T // 4, DIM), lambda g: (g, 0)),
            pl.BlockSpec((T // 4, 1), lambda g: (g, 0)),
        ],
        out_specs=pl.BlockSpec((T // 4, DIM), lambda g: (g, 0)),
        out_shape=jax.ShapeDtypeStruct((T, DIM), jnp.float32),
    )(next_r, outa, outb, w_used)
    return out


# speedup vs baseline: 1.0622x; 1.0375x over previous
import jax
import jax.numpy as jnp
from jax import lax
from jax.experimental import pallas as pl
from jax.experimental.pallas import tpu as pltpu
from jax.experimental.pallas import tpu_sc as plsc

T = 4096
DIM = 1024
DFF = 2048
E = 8
K = 2

TILE_A = 512
TILE_F = 256
NTILES = (T * K) // TILE_F + E
NPAD = NTILES * TILE_F
DFFB = 2048
NDFF = DFF // DFFB
TE_LEN = 80

RCH = 1024
NRCH = (T * K) // RCH

NC = 2
NS = 16
NW = NC * NS
TPW = T // NW
CH = 32
NCH = TPW // CH

_MESH = plsc.VectorSubcoreMesh(core_axis_name="c", subcore_axis_name="s",
                               num_cores=NC, num_subcores=NS)


def _route_body(x_ref, gw_ref, ti0_ref, ti1_ref, tw_ref):
    xb = x_ref[...]
    logits = lax.dot_general(xb, gw_ref[...], (((1,), (1,)), ((), ())),
                             preferred_element_type=jnp.float32)
    eids = lax.broadcasted_iota(jnp.int32, (1, E), 1)
    v1 = jnp.max(logits, axis=1, keepdims=True)
    i1 = jnp.argmax(logits, axis=1, keepdims=True).astype(jnp.int32)
    masked = jnp.where(eids == i1, -jnp.inf, logits)
    v2 = jnp.max(masked, axis=1, keepdims=True)
    i2 = jnp.argmax(masked, axis=1, keepdims=True).astype(jnp.int32)
    ti0_ref[...] = i1
    ti1_ref[...] = i2
    e2 = jnp.exp(v2 - v1)
    denom = 1.0 + e2
    tw_ref[...] = jnp.concatenate([1.0 / denom, e2 / denom], axis=1)


def _rank_body(ti0_ref, ti1_ref, pe_ref, po_ref, te_ref):
    eids = lax.broadcasted_iota(jnp.int32, (1, E), 1)
    oh0 = jnp.where(ti0_ref[...] == eids, 1.0, 0.0)
    oh1 = jnp.where(ti1_ref[...] == eids, 1.0, 0.0)
    ohi = jnp.concatenate(
        [oh0.reshape(T, 1, E), oh1.reshape(T, 1, E)], axis=1
    ).reshape(2 * T, E)

    tot = jnp.sum(ohi, axis=0, keepdims=True)
    padded = jnp.floor((tot + (TILE_F - 1)) * (1.0 / TILE_F)) * TILE_F
    r8 = lax.broadcasted_iota(jnp.int32, (E, E), 0)
    c8 = lax.broadcasted_iota(jnp.int32, (E, E), 1)
    u8 = jnp.where(r8 < c8, 1.0, 0.0)
    pad_base = lax.dot_general(padded, u8, (((1,), (0,)), ((), ())),
                               preferred_element_type=jnp.float32)

    rr = lax.broadcasted_iota(jnp.int32, (RCH, RCH), 0)
    cc = lax.broadcasted_iota(jnp.int32, (RCH, RCH), 1)
    lt = jnp.where(cc < rr, 1.0, 0.0)

    carry = jnp.zeros((1, E), jnp.float32)
    for c in range(NRCH):
        chunk = ohi[c * RCH:(c + 1) * RCH]
        within = lax.dot_general(lt, chunk, (((1,), (0,)), ((), ())),
                                 preferred_element_type=jnp.float32)
        base_vec = pad_base + carry
        basep = jnp.sum(chunk * base_vec, axis=1, keepdims=True)
        withinp = jnp.sum(within * chunk, axis=1, keepdims=True)
        pos = (basep + withinp).astype(jnp.int32).reshape(RCH // K, K)
        pe_ref[pl.ds(c * (RCH // K), RCH // K), :] = pos[:, 0:1]
        po_ref[pl.ds(c * (RCH // K), RCH // K), :] = pos[:, 1:2]
        carry = carry + jnp.sum(chunk, axis=0, keepdims=True)

    gi = lax.broadcasted_iota(jnp.int32, (TE_LEN, E), 0)
    ge = jnp.where((gi * TILE_F).astype(jnp.float32) >= pad_base,
                   1.0, 0.0)
    te_ref[...] = (jnp.sum(ge, axis=1, keepdims=True)
                   - 1.0).astype(jnp.int32)


def _sc_dispatch_body(x_hbm, pe_hbm, po_hbm, xs_hbm, pe2_v, po2_v, xbuf_v):
    wid = lax.axis_index("s") * NC + lax.axis_index("c")
    base = wid * TPW
    for ch in range(NCH):
        pltpu.sync_copy(pe_hbm.at[pl.ds(base + ch * CH, CH)], pe2_v.at[ch])
        pltpu.sync_copy(po_hbm.at[pl.ds(base + ch * CH, CH)], po2_v.at[ch])
    for ch in range(NCH):
        pltpu.sync_copy(x_hbm.at[pl.ds(base + ch * CH, CH)], xbuf_v)
        pltpu.sync_copy(xbuf_v, xs_hbm.at[pe2_v.at[ch]])
        pltpu.sync_copy(xbuf_v, xs_hbm.at[po2_v.at[ch]])


def _ffn_body(te_ref, x_ref, w1_ref, w3_ref, w2_ref, out_ref):
    f_idx = pl.program_id(1)

    @pl.when(f_idx == 0)
    def _():
        out_ref[...] = jnp.zeros_like(out_ref)

    xb = x_ref[...]
    h1 = lax.dot_general(xb, w1_ref[0], (((1,), (1,)), ((), ())),
                         preferred_element_type=jnp.float32)
    h3 = lax.dot_general(xb, w3_ref[0], (((1,), (1,)), ((), ())),
                         preferred_element_type=jnp.float32)
    hg = h1 * lax.logistic(h1) * h3
    out_ref[...] += lax.dot_general(hg, w2_ref[0], (((1,), (0,)), ((), ())),
                                    preferred_element_type=jnp.float32)


def _sc_unsort_body(os_hbm, pe_hbm, po_hbm, outa_hbm, outb_hbm,
                    idx2_v, gbuf_v):
    wid = lax.axis_index("s") * NC + lax.axis_index("c")
    base = wid * TPW
    for p_hbm, o_hbm in ((pe_hbm, outa_hbm), (po_hbm, outb_hbm)):
        for ch in range(NCH):
            t0 = base + ch * CH
            pltpu.sync_copy(p_hbm.at[pl.ds(t0, CH)], idx2_v.at[ch])
            pltpu.sync_copy(os_hbm.at[idx2_v.at[ch]], gbuf_v)
            pltpu.sync_copy(gbuf_v, o_hbm.at[pl.ds(t0, CH)])


def _combine_body(nr_ref, a_ref, b_ref, w_ref, out_ref):
    out_ref[...] = nr_ref[...] + w_ref[...] * (a_ref[...] + b_ref[...])


_sc_dispatch = pl.kernel(
    _sc_dispatch_body,
    out_type=jax.ShapeDtypeStruct((NPAD, DIM), jnp.float32),
    mesh=_MESH,
    scratch_types=[pltpu.VMEM((NCH, CH), jnp.int32),
                   pltpu.VMEM((NCH, CH), jnp.int32),
                   pltpu.VMEM((CH, DIM), jnp.float32)])

_sc_unsort = pl.kernel(
    _sc_unsort_body,
    out_type=[jax.ShapeDtypeStruct((T, DIM), jnp.float32),
              jax.ShapeDtypeStruct((T, DIM), jnp.float32)],
    mesh=_MESH,
    scratch_types=[pltpu.VMEM((NCH, CH), jnp.int32),
                   pltpu.VMEM((CH, DIM), jnp.float32)])


@jax.jit
def kernel(x, next_r, gate_w, w1, w2, w3):
    ti0, ti1, tw = pl.pallas_call(
        _route_body,
        grid=(T // TILE_A,),
        in_specs=[
            pl.BlockSpec((TILE_A, DIM), lambda g: (g, 0)),
            pl.BlockSpec((E, DIM), lambda g: (0, 0)),
        ],
        out_specs=[
            pl.BlockSpec((TILE_A, 1), lambda g: (g, 0)),
            pl.BlockSpec((TILE_A, 1), lambda g: (g, 0)),
            pl.BlockSpec((TILE_A, K), lambda g: (g, 0)),
        ],
        out_shape=[
            jax.ShapeDtypeStruct((T, 1), jnp.int32),
            jax.ShapeDtypeStruct((T, 1), jnp.int32),
            jax.ShapeDtypeStruct((T, K), jnp.float32),
        ],
    )(x, gate_w)

    w_used = tw[: T // K].reshape(T, 1)

    pe2, po2, te2 = pl.pallas_call(
        _rank_body,
        out_shape=[
            jax.ShapeDtypeStruct((T, 1), jnp.int32),
            jax.ShapeDtypeStruct((T, 1), jnp.int32),
            jax.ShapeDtypeStruct((TE_LEN, 1), jnp.int32),
        ],
    )(ti0, ti1)

    pe = pe2.reshape(T)
    po = po2.reshape(T)
    te = te2.reshape(TE_LEN)

    xs = _sc_dispatch(x, pe, po)

    outs = pl.pallas_call(
        _ffn_body,
        grid_spec=pltpu.PrefetchScalarGridSpec(
            num_scalar_prefetch=1,
            grid=(NTILES, NDFF),
            in_specs=[
                pl.BlockSpec((TILE_F, DIM), lambda g, f, te_r: (g, 0)),
                pl.BlockSpec((1, DFFB, DIM),
                             lambda g, f, te_r: (te_r[g], f, 0)),
                pl.BlockSpec((1, DFFB, DIM),
                             lambda g, f, te_r: (te_r[g], f, 0)),
                pl.BlockSpec((1, DFFB, DIM),
                             lambda g, f, te_r: (te_r[g], f, 0)),
            ],
            out_specs=pl.BlockSpec((TILE_F, DIM), lambda g, f, te_r: (g, 0)),
        ),
        out_shape=jax.ShapeDtypeStruct((NPAD, DIM), jnp.float32),
        compiler_params=pltpu.CompilerParams(
            vmem_limit_bytes=63 * 1024 * 1024),
    )(te, xs, w1, w3, w2)

    outa, outb = _sc_unsort(outs, pe, po)

    out = pl.pallas_call(
        _combine_body,
        grid=(4,),
        in_specs=[
            pl.BlockSpec((T // 4, DIM), lambda g: (g, 0)),
            pl.BlockSpec((T // 4, DIM), lambda g: (g, 0)),
            pl.BlockSpec((T // 4, DIM), lambda g: (g, 0)),
            pl.BlockSpec((T // 4, 1), lambda g: (g, 0)),
        ],
        out_specs=pl.BlockSpec((T // 4, DIM), lambda g: (g, 0)),
        out_shape=jax.ShapeDtypeStruct((T, DIM), jnp.float32),
    )(next_r, outa, outb, w_used)
    return out


# speedup vs baseline: 1.1269x; 1.0609x over previous
import jax
import jax.numpy as jnp
from jax import lax
from jax.experimental import pallas as pl
from jax.experimental.pallas import tpu as pltpu
from jax.experimental.pallas import tpu_sc as plsc

T = 4096
DIM = 1024
DFF = 2048
E = 8
K = 2

TILE_A = 512
TILE_F = 256
NTILES = (T * K) // TILE_F + E
NPAD = NTILES * TILE_F
DFFB = 2048
NDFF = DFF // DFFB
TE_LEN = 80

RCH = 1024
NRCH = (T * K) // RCH

NC = 2
NS = 16
NW = NC * NS
TPW = T // NW
CH = 32
NCH = TPW // CH

_MESH = plsc.VectorSubcoreMesh(core_axis_name="c", subcore_axis_name="s",
                               num_cores=NC, num_subcores=NS)


def _route_rank_body(x_ref, gw_ref, tw_ref, pe_ref, po_ref, te_ref, act_ref,
                     ohi_s):
    g = pl.program_id(0)
    xb = x_ref[...]
    logits = lax.dot_general(xb, gw_ref[...], (((1,), (1,)), ((), ())),
                             preferred_element_type=jnp.float32)
    eids = lax.broadcasted_iota(jnp.int32, (1, E), 1)
    v1 = jnp.max(logits, axis=1, keepdims=True)
    i1 = jnp.argmax(logits, axis=1, keepdims=True).astype(jnp.int32)
    masked = jnp.where(eids == i1, -jnp.inf, logits)
    v2 = jnp.max(masked, axis=1, keepdims=True)
    i2 = jnp.argmax(masked, axis=1, keepdims=True).astype(jnp.int32)
    e2 = jnp.exp(v2 - v1)
    denom = 1.0 + e2
    tw_ref[...] = jnp.concatenate([1.0 / denom, e2 / denom], axis=1)

    oh0 = jnp.where(i1 == eids, 1.0, 0.0)
    oh1 = jnp.where(i2 == eids, 1.0, 0.0)
    ohg = jnp.concatenate(
        [oh0.reshape(TILE_A, 1, E), oh1.reshape(TILE_A, 1, E)], axis=1
    ).reshape(2 * TILE_A, E)
    ohi_s[pl.ds(g * 2 * TILE_A, 2 * TILE_A), :] = ohg

    @pl.when(g == (T // TILE_A) - 1)
    def _():
        ohi = ohi_s[...]
        tot = jnp.sum(ohi, axis=0, keepdims=True)
        padded = jnp.floor((tot + (TILE_F - 1)) * (1.0 / TILE_F)) * TILE_F
        r8 = lax.broadcasted_iota(jnp.int32, (E, E), 0)
        c8 = lax.broadcasted_iota(jnp.int32, (E, E), 1)
        u8 = jnp.where(r8 < c8, 1.0, 0.0)
        pad_base = lax.dot_general(padded, u8, (((1,), (0,)), ((), ())),
                                   preferred_element_type=jnp.float32)
        total_rows = jnp.sum(padded)

        rr = lax.broadcasted_iota(jnp.int32, (RCH, RCH), 0)
        cc = lax.broadcasted_iota(jnp.int32, (RCH, RCH), 1)
        lt = jnp.where(cc < rr, 1.0, 0.0)

        carry = jnp.zeros((1, E), jnp.float32)
        for c in range(NRCH):
            chunk = ohi[c * RCH:(c + 1) * RCH]
            within = lax.dot_general(lt, chunk, (((1,), (0,)), ((), ())),
                                     preferred_element_type=jnp.float32)
            base_vec = pad_base + carry
            basep = jnp.sum(chunk * base_vec, axis=1, keepdims=True)
            withinp = jnp.sum(within * chunk, axis=1, keepdims=True)
            pos = (basep + withinp).astype(jnp.int32).reshape(RCH // K, K)
            pe_ref[pl.ds(c * (RCH // K), RCH // K), :] = pos[:, 0:1]
            po_ref[pl.ds(c * (RCH // K), RCH // K), :] = pos[:, 1:2]
            carry = carry + jnp.sum(chunk, axis=0, keepdims=True)

        gi = lax.broadcasted_iota(jnp.int32, (TE_LEN, E), 0)
        ge = jnp.where((gi * TILE_F).astype(jnp.float32) >= pad_base,
                       1.0, 0.0)
        te_ref[...] = (jnp.sum(ge, axis=1, keepdims=True)
                       - 1.0).astype(jnp.int32)
        act_ref[...] = jnp.where(
            (gi[:, 0:1] * TILE_F).astype(jnp.float32) < total_rows,
            1, 0).astype(jnp.int32)


def _sc_dispatch_body(x_hbm, pe_hbm, po_hbm, xs_hbm, pe2_v, po2_v, xbuf_v):
    wid = lax.axis_index("s") * NC + lax.axis_index("c")
    base = wid * TPW
    for ch in range(NCH):
        pltpu.sync_copy(pe_hbm.at[pl.ds(base + ch * CH, CH)], pe2_v.at[ch])
        pltpu.sync_copy(po_hbm.at[pl.ds(base + ch * CH, CH)], po2_v.at[ch])
    for ch in range(NCH):
        pltpu.sync_copy(x_hbm.at[pl.ds(base + ch * CH, CH)], xbuf_v)
        pltpu.sync_copy(xbuf_v, xs_hbm.at[pe2_v.at[ch]])
        pltpu.sync_copy(xbuf_v, xs_hbm.at[po2_v.at[ch]])


def _ffn_body(te_ref, act_ref, x_ref, w1_ref, w3_ref, w2_ref, out_ref):
    g_idx = pl.program_id(0)
    f_idx = pl.program_id(1)

    @pl.when(act_ref[g_idx] == 1)
    def _():
        @pl.when(f_idx == 0)
        def _():
            out_ref[...] = jnp.zeros_like(out_ref)

        xb = x_ref[...]
        h1 = lax.dot_general(xb, w1_ref[0], (((1,), (1,)), ((), ())),
                             preferred_element_type=jnp.float32)
        h3 = lax.dot_general(xb, w3_ref[0], (((1,), (1,)), ((), ())),
                             preferred_element_type=jnp.float32)
        hg = h1 * lax.logistic(h1) * h3
        out_ref[...] += lax.dot_general(hg, w2_ref[0],
                                        (((1,), (0,)), ((), ())),
                                        preferred_element_type=jnp.float32)


def _sc_unsort_body(os_hbm, pe_hbm, po_hbm, outa_hbm, outb_hbm,
                    idx2_v, gbuf_v):
    wid = lax.axis_index("s") * NC + lax.axis_index("c")
    base = wid * TPW
    for p_hbm, o_hbm in ((pe_hbm, outa_hbm), (po_hbm, outb_hbm)):
        for ch in range(NCH):
            t0 = base + ch * CH
            pltpu.sync_copy(p_hbm.at[pl.ds(t0, CH)], idx2_v.at[ch])
            pltpu.sync_copy(os_hbm.at[idx2_v.at[ch]], gbuf_v)
            pltpu.sync_copy(gbuf_v, o_hbm.at[pl.ds(t0, CH)])


def _combine_body(nr_ref, a_ref, b_ref, w_ref, out_ref):
    out_ref[...] = nr_ref[...] + w_ref[...] * (a_ref[...] + b_ref[...])


_sc_dispatch = pl.kernel(
    _sc_dispatch_body,
    out_type=jax.ShapeDtypeStruct((NPAD, DIM), jnp.float32),
    mesh=_MESH,
    scratch_types=[pltpu.VMEM((NCH, CH), jnp.int32),
                   pltpu.VMEM((NCH, CH), jnp.int32),
                   pltpu.VMEM((CH, DIM), jnp.float32)])

_sc_unsort = pl.kernel(
    _sc_unsort_body,
    out_type=[jax.ShapeDtypeStruct((T, DIM), jnp.float32),
              jax.ShapeDtypeStruct((T, DIM), jnp.float32)],
    mesh=_MESH,
    scratch_types=[pltpu.VMEM((NCH, CH), jnp.int32),
                   pltpu.VMEM((CH, DIM), jnp.float32)])


@jax.jit
def kernel(x, next_r, gate_w, w1, w2, w3):
    tw, pe2, po2, te2, act2 = pl.pallas_call(
        _route_rank_body,
        grid=(T // TILE_A,),
        in_specs=[
            pl.BlockSpec((TILE_A, DIM), lambda g: (g, 0)),
            pl.BlockSpec((E, DIM), lambda g: (0, 0)),
        ],
        out_specs=[
            pl.BlockSpec((TILE_A, K), lambda g: (g, 0)),
            pl.BlockSpec((T, 1), lambda g: (0, 0)),
            pl.BlockSpec((T, 1), lambda g: (0, 0)),
            pl.BlockSpec((TE_LEN, 1), lambda g: (0, 0)),
            pl.BlockSpec((TE_LEN, 1), lambda g: (0, 0)),
        ],
        out_shape=[
            jax.ShapeDtypeStruct((T, K), jnp.float32),
            jax.ShapeDtypeStruct((T, 1), jnp.int32),
            jax.ShapeDtypeStruct((T, 1), jnp.int32),
            jax.ShapeDtypeStruct((TE_LEN, 1), jnp.int32),
            jax.ShapeDtypeStruct((TE_LEN, 1), jnp.int32),
        ],
        scratch_shapes=[
            pltpu.VMEM((2 * T, E), jnp.float32),
        ],
    )(x, gate_w)

    w_used = tw[: T // K].reshape(T, 1)

    pe = pe2.reshape(T)
    po = po2.reshape(T)
    te = te2.reshape(TE_LEN)
    act = act2.reshape(TE_LEN)

    xs = _sc_dispatch(x, pe, po)

    outs = pl.pallas_call(
        _ffn_body,
        grid_spec=pltpu.PrefetchScalarGridSpec(
            num_scalar_prefetch=2,
            grid=(NTILES, NDFF),
            in_specs=[
                pl.BlockSpec((TILE_F, DIM), lambda g, f, te_r, a_r: (g, 0)),
                pl.BlockSpec((1, DFFB, DIM),
                             lambda g, f, te_r, a_r: (te_r[g], f, 0)),
                pl.BlockSpec((1, DFFB, DIM),
                             lambda g, f, te_r, a_r: (te_r[g], f, 0)),
                pl.BlockSpec((1, DFFB, DIM),
                             lambda g, f, te_r, a_r: (te_r[g], f, 0)),
            ],
            out_specs=pl.BlockSpec((TILE_F, DIM),
                                   lambda g, f, te_r, a_r: (g, 0)),
        ),
        out_shape=jax.ShapeDtypeStruct((NPAD, DIM), jnp.float32),
        compiler_params=pltpu.CompilerParams(
            vmem_limit_bytes=63 * 1024 * 1024),
    )(te, act, xs, w1, w3, w2)

    outa, outb = _sc_unsort(outs, pe, po)

    out = pl.pallas_call(
        _combine_body,
        grid=(4,),
        in_specs=[
            pl.BlockSpec((T // 4, DIM), lambda g: (g, 0)),
            pl.BlockSpec((T // 4, DIM), lambda g: (g, 0)),
            pl.BlockSpec((T // 4, DIM), lambda g: (g, 0)),
            pl.BlockSpec((T // 4, 1), lambda g: (g, 0)),
        ],
        out_specs=pl.BlockSpec((T // 4, DIM), lambda g: (g, 0)),
        out_shape=jax.ShapeDtypeStruct((T, DIM), jnp.float32),
    )(next_r, outa, outb, w_used)
    return out


# speedup vs baseline: 1.1427x; 1.0141x over previous
import jax
import jax.numpy as jnp
from jax import lax
from jax.experimental import pallas as pl
from jax.experimental.pallas import tpu as pltpu
from jax.experimental.pallas import tpu_sc as plsc

T = 4096
DIM = 1024
DFF = 2048
E = 8
K = 2

TILE_A = 512
TILE_F = 256
NTILES = (T * K) // TILE_F + E
NPAD = NTILES * TILE_F
DFFB = 2048
NDFF = DFF // DFFB
TE_LEN = 80

RCH = 1024
NRCH = (T * K) // RCH

NC = 2
NS = 16
NW = NC * NS
TPW = T // NW
CH = 32
NCH = TPW // CH

_MESH = plsc.VectorSubcoreMesh(core_axis_name="c", subcore_axis_name="s",
                               num_cores=NC, num_subcores=NS)


def _route_rank_body(x_ref, gw_ref, tw_ref, pe_ref, po_ref, te_ref, act_ref,
                     ohi_s):
    g = pl.program_id(0)
    xb = x_ref[...]
    logits = lax.dot_general(xb, gw_ref[...], (((1,), (1,)), ((), ())),
                             preferred_element_type=jnp.float32)
    eids = lax.broadcasted_iota(jnp.int32, (1, E), 1)
    v1 = jnp.max(logits, axis=1, keepdims=True)
    i1 = jnp.argmax(logits, axis=1, keepdims=True).astype(jnp.int32)
    masked = jnp.where(eids == i1, -jnp.inf, logits)
    v2 = jnp.max(masked, axis=1, keepdims=True)
    i2 = jnp.argmax(masked, axis=1, keepdims=True).astype(jnp.int32)
    e2 = jnp.exp(v2 - v1)
    denom = 1.0 + e2
    tw_ref[...] = jnp.concatenate([1.0 / denom, e2 / denom], axis=1)

    oh0 = jnp.where(i1 == eids, 1.0, 0.0)
    oh1 = jnp.where(i2 == eids, 1.0, 0.0)
    ohg = jnp.concatenate(
        [oh0.reshape(TILE_A, 1, E), oh1.reshape(TILE_A, 1, E)], axis=1
    ).reshape(2 * TILE_A, E)
    ohi_s[pl.ds(g * 2 * TILE_A, 2 * TILE_A), :] = ohg

    @pl.when(g == (T // TILE_A) - 1)
    def _():
        ohi = ohi_s[...]
        tot = jnp.sum(ohi, axis=0, keepdims=True)
        padded = jnp.floor((tot + (TILE_F - 1)) * (1.0 / TILE_F)) * TILE_F
        r8 = lax.broadcasted_iota(jnp.int32, (E, E), 0)
        c8 = lax.broadcasted_iota(jnp.int32, (E, E), 1)
        u8 = jnp.where(r8 < c8, 1.0, 0.0)
        pad_base = lax.dot_general(padded, u8, (((1,), (0,)), ((), ())),
                                   preferred_element_type=jnp.float32)
        total_rows = jnp.sum(padded)

        rr = lax.broadcasted_iota(jnp.int32, (RCH, RCH), 0)
        cc = lax.broadcasted_iota(jnp.int32, (RCH, RCH), 1)
        lt = jnp.where(cc < rr, 1.0, 0.0)

        carry = jnp.zeros((1, E), jnp.float32)
        for c in range(NRCH):
            chunk = ohi[c * RCH:(c + 1) * RCH]
            within = lax.dot_general(lt, chunk, (((1,), (0,)), ((), ())),
                                     preferred_element_type=jnp.float32)
            base_vec = pad_base + carry
            basep = jnp.sum(chunk * base_vec, axis=1, keepdims=True)
            withinp = jnp.sum(within * chunk, axis=1, keepdims=True)
            pos = (basep + withinp).astype(jnp.int32).reshape(RCH // K, K)
            pe_ref[pl.ds(c * (RCH // K), RCH // K), :] = pos[:, 0:1]
            po_ref[pl.ds(c * (RCH // K), RCH // K), :] = pos[:, 1:2]
            carry = carry + jnp.sum(chunk, axis=0, keepdims=True)

        gi = lax.broadcasted_iota(jnp.int32, (TE_LEN, E), 0)
        ge = jnp.where((gi * TILE_F).astype(jnp.float32) >= pad_base,
                       1.0, 0.0)
        te_ref[...] = (jnp.sum(ge, axis=1, keepdims=True)
                       - 1.0).astype(jnp.int32)
        act_ref[...] = jnp.where(
            (gi[:, 0:1] * TILE_F).astype(jnp.float32) < total_rows,
            1, 0).astype(jnp.int32)


def _sc_dispatch_body(x_hbm, pe_hbm, po_hbm, xs_hbm, pe2_v, po2_v, xbuf_v):
    wid = lax.axis_index("s") * NC + lax.axis_index("c")
    base = wid * TPW
    pltpu.sync_copy(pe_hbm.at[pl.ds(base, TPW)], pe2_v)
    pltpu.sync_copy(po_hbm.at[pl.ds(base, TPW)], po2_v)
    for ch in range(NCH):
        pltpu.sync_copy(x_hbm.at[pl.ds(base + ch * CH, CH)], xbuf_v)
        pltpu.sync_copy(xbuf_v, xs_hbm.at[pe2_v.at[pl.ds(ch * CH, CH)]])
        pltpu.sync_copy(xbuf_v, xs_hbm.at[po2_v.at[pl.ds(ch * CH, CH)]])


def _ffn_body(te_ref, act_ref, x_ref, w1_ref, w3_ref, w2_ref, out_ref):
    g_idx = pl.program_id(0)
    f_idx = pl.program_id(1)

    @pl.when(act_ref[g_idx] == 1)
    def _():
        @pl.when(f_idx == 0)
        def _():
            out_ref[...] = jnp.zeros_like(out_ref)

        xb = x_ref[...]
        h1 = lax.dot_general(xb, w1_ref[0], (((1,), (1,)), ((), ())),
                             preferred_element_type=jnp.float32)
        h3 = lax.dot_general(xb, w3_ref[0], (((1,), (1,)), ((), ())),
                             preferred_element_type=jnp.float32)
        hg = h1 * lax.logistic(h1) * h3
        out_ref[...] += lax.dot_general(hg, w2_ref[0],
                                        (((1,), (0,)), ((), ())),
                                        preferred_element_type=jnp.float32)


def _sc_unsort_body(os_hbm, pe_hbm, po_hbm, outa_hbm, outb_hbm,
                    idx2_v, gbuf_v):
    wid = lax.axis_index("s") * NC + lax.axis_index("c")
    base = wid * TPW
    for p_hbm, o_hbm in ((pe_hbm, outa_hbm), (po_hbm, outb_hbm)):
        pltpu.sync_copy(p_hbm.at[pl.ds(base, TPW)], idx2_v)
        for ch in range(NCH):
            t0 = base + ch * CH
            pltpu.sync_copy(os_hbm.at[idx2_v.at[pl.ds(ch * CH, CH)]], gbuf_v)
            pltpu.sync_copy(gbuf_v, o_hbm.at[pl.ds(t0, CH)])


def _combine_body(nr_ref, a_ref, b_ref, w_ref, out_ref):
    out_ref[...] = nr_ref[...] + w_ref[...] * (a_ref[...] + b_ref[...])


_sc_dispatch = pl.kernel(
    _sc_dispatch_body,
    out_type=jax.ShapeDtypeStruct((NPAD, DIM), jnp.float32),
    mesh=_MESH,
    scratch_types=[pltpu.VMEM((TPW,), jnp.int32),
                   pltpu.VMEM((TPW,), jnp.int32),
                   pltpu.VMEM((CH, DIM), jnp.float32)])

_sc_unsort = pl.kernel(
    _sc_unsort_body,
    out_type=[jax.ShapeDtypeStruct((T, DIM), jnp.float32),
              jax.ShapeDtypeStruct((T, DIM), jnp.float32)],
    mesh=_MESH,
    scratch_types=[pltpu.VMEM((TPW,), jnp.int32),
                   pltpu.VMEM((CH, DIM), jnp.float32)])


@jax.jit
def kernel(x, next_r, gate_w, w1, w2, w3):
    tw, pe2, po2, te2, act2 = pl.pallas_call(
        _route_rank_body,
        grid=(T // TILE_A,),
        in_specs=[
            pl.BlockSpec((TILE_A, DIM), lambda g: (g, 0)),
            pl.BlockSpec((E, DIM), lambda g: (0, 0)),
        ],
        out_specs=[
            pl.BlockSpec((TILE_A, K), lambda g: (g, 0)),
            pl.BlockSpec((T, 1), lambda g: (0, 0)),
            pl.BlockSpec((T, 1), lambda g: (0, 0)),
            pl.BlockSpec((TE_LEN, 1), lambda g: (0, 0)),
            pl.BlockSpec((TE_LEN, 1), lambda g: (0, 0)),
        ],
        out_shape=[
            jax.ShapeDtypeStruct((T, K), jnp.float32),
            jax.ShapeDtypeStruct((T, 1), jnp.int32),
            jax.ShapeDtypeStruct((T, 1), jnp.int32),
            jax.ShapeDtypeStruct((TE_LEN, 1), jnp.int32),
            jax.ShapeDtypeStruct((TE_LEN, 1), jnp.int32),
        ],
        scratch_shapes=[
            pltpu.VMEM((2 * T, E), jnp.float32),
        ],
    )(x, gate_w)

    w_used = tw[: T // K].reshape(T, 1)

    pe = pe2.reshape(T)
    po = po2.reshape(T)
    te = te2.reshape(TE_LEN)
    act = act2.reshape(TE_LEN)

    xs = _sc_dispatch(x, pe, po)

    outs = pl.pallas_call(
        _ffn_body,
        grid_spec=pltpu.PrefetchScalarGridSpec(
            num_scalar_prefetch=2,
            grid=(NTILES, NDFF),
            in_specs=[
                pl.BlockSpec((TILE_F, DIM),
                             lambda g, f, te_r, a_r: (g * a_r[g], 0)),
                pl.BlockSpec((1, DFFB, DIM),
                             lambda g, f, te_r, a_r: (te_r[g], f, 0)),
                pl.BlockSpec((1, DFFB, DIM),
                             lambda g, f, te_r, a_r: (te_r[g], f, 0)),
                pl.BlockSpec((1, DFFB, DIM),
                             lambda g, f, te_r, a_r: (te_r[g], f, 0)),
            ],
            out_specs=pl.BlockSpec((TILE_F, DIM),
                                   lambda g, f, te_r, a_r: (g, 0)),
        ),
        out_shape=jax.ShapeDtypeStruct((NPAD, DIM), jnp.float32),
        compiler_params=pltpu.CompilerParams(
            vmem_limit_bytes=63 * 1024 * 1024),
    )(te, act, xs, w1, w3, w2)

    outa, outb = _sc_unsort(outs, pe, po)

    out = pl.pallas_call(
        _combine_body,
        grid=(4,),
        in_specs=[
            pl.BlockSpec((T // 4, DIM), lambda g: (g, 0)),
            pl.BlockSpec((T // 4, DIM), lambda g: (g, 0)),
            pl.BlockSpec((T // 4, DIM), lambda g: (g, 0)),
            pl.BlockSpec((T // 4, 1), lambda g: (g, 0)),
        ],
        out_specs=pl.BlockSpec((T // 4, DIM), lambda g: (g, 0)),
        out_shape=jax.ShapeDtypeStruct((T, DIM), jnp.float32),
    )(next_r, outa, outb, w_used)
    return out


# speedup vs baseline: 1.1656x; 1.0200x over previous
import jax
import jax.numpy as jnp
from jax import lax
from jax.experimental import pallas as pl
from jax.experimental.pallas import tpu as pltpu
from jax.experimental.pallas import tpu_sc as plsc

T = 4096
DIM = 1024
DFF = 2048
E = 8
K = 2

TILE_A = 512
TILE_F = 256
NTILES = (T * K) // TILE_F + E
NPAD = NTILES * TILE_F
DFFB = 2048
NDFF = DFF // DFFB
TE_LEN = 80

RCH = 1024
NRCH = (T * K) // RCH

NC = 2
NS = 16
NW = NC * NS
TPW = T // NW
CH = 64
NCH = TPW // CH

_MESH = plsc.VectorSubcoreMesh(core_axis_name="c", subcore_axis_name="s",
                               num_cores=NC, num_subcores=NS)


def _route_rank_body(x_ref, gw_ref, tw_ref, pe_ref, po_ref, te_ref, act_ref,
                     ohi_s):
    g = pl.program_id(0)
    xb = x_ref[...]
    logits = lax.dot_general(xb, gw_ref[...], (((1,), (1,)), ((), ())),
                             preferred_element_type=jnp.float32)
    eids = lax.broadcasted_iota(jnp.int32, (1, E), 1)
    v1 = jnp.max(logits, axis=1, keepdims=True)
    i1 = jnp.argmax(logits, axis=1, keepdims=True).astype(jnp.int32)
    masked = jnp.where(eids == i1, -jnp.inf, logits)
    v2 = jnp.max(masked, axis=1, keepdims=True)
    i2 = jnp.argmax(masked, axis=1, keepdims=True).astype(jnp.int32)
    e2 = jnp.exp(v2 - v1)
    denom = 1.0 + e2
    tw_ref[...] = jnp.concatenate([1.0 / denom, e2 / denom], axis=1)

    oh0 = jnp.where(i1 == eids, 1.0, 0.0)
    oh1 = jnp.where(i2 == eids, 1.0, 0.0)
    ohg = jnp.concatenate(
        [oh0.reshape(TILE_A, 1, E), oh1.reshape(TILE_A, 1, E)], axis=1
    ).reshape(2 * TILE_A, E)
    ohi_s[pl.ds(g * 2 * TILE_A, 2 * TILE_A), :] = ohg

    @pl.when(g == (T // TILE_A) - 1)
    def _():
        ohi = ohi_s[...]
        tot = jnp.sum(ohi, axis=0, keepdims=True)
        padded = jnp.floor((tot + (TILE_F - 1)) * (1.0 / TILE_F)) * TILE_F
        r8 = lax.broadcasted_iota(jnp.int32, (E, E), 0)
        c8 = lax.broadcasted_iota(jnp.int32, (E, E), 1)
        u8 = jnp.where(r8 < c8, 1.0, 0.0)
        pad_base = lax.dot_general(padded, u8, (((1,), (0,)), ((), ())),
                                   preferred_element_type=jnp.float32)
        total_rows = jnp.sum(padded)

        rr = lax.broadcasted_iota(jnp.int32, (RCH, RCH), 0)
        cc = lax.broadcasted_iota(jnp.int32, (RCH, RCH), 1)
        lt = jnp.where(cc < rr, 1.0, 0.0)

        carry = jnp.zeros((1, E), jnp.float32)
        for c in range(NRCH):
            chunk = ohi[c * RCH:(c + 1) * RCH]
            within = lax.dot_general(lt, chunk, (((1,), (0,)), ((), ())),
                                     preferred_element_type=jnp.float32)
            base_vec = pad_base + carry
            basep = jnp.sum(chunk * base_vec, axis=1, keepdims=True)
            withinp = jnp.sum(within * chunk, axis=1, keepdims=True)
            pos = (basep + withinp).astype(jnp.int32).reshape(RCH // K, K)
            pe_ref[pl.ds(c * (RCH // K), RCH // K), :] = pos[:, 0:1]
            po_ref[pl.ds(c * (RCH // K), RCH // K), :] = pos[:, 1:2]
            carry = carry + jnp.sum(chunk, axis=0, keepdims=True)

        gi = lax.broadcasted_iota(jnp.int32, (TE_LEN, E), 0)
        ge = jnp.where((gi * TILE_F).astype(jnp.float32) >= pad_base,
                       1.0, 0.0)
        te_ref[...] = (jnp.sum(ge, axis=1, keepdims=True)
                       - 1.0).astype(jnp.int32)
        act_ref[...] = jnp.where(
            (gi[:, 0:1] * TILE_F).astype(jnp.float32) < total_rows,
            1, 0).astype(jnp.int32)


def _sc_dispatch_body(x_hbm, pe_hbm, po_hbm, xs_hbm, pe2_v, po2_v, xbuf_v):
    wid = lax.axis_index("s") * NC + lax.axis_index("c")
    base = wid * TPW
    pltpu.sync_copy(pe_hbm.at[pl.ds(base, TPW)], pe2_v)
    pltpu.sync_copy(po_hbm.at[pl.ds(base, TPW)], po2_v)
    for ch in range(NCH):
        pltpu.sync_copy(x_hbm.at[pl.ds(base + ch * CH, CH)], xbuf_v)
        pltpu.sync_copy(xbuf_v, xs_hbm.at[pe2_v.at[pl.ds(ch * CH, CH)]])
        pltpu.sync_copy(xbuf_v, xs_hbm.at[po2_v.at[pl.ds(ch * CH, CH)]])


def _ffn_body(te_ref, act_ref, x_ref, w1_ref, w3_ref, w2_ref, out_ref):
    g_idx = pl.program_id(0)
    f_idx = pl.program_id(1)

    @pl.when(act_ref[g_idx] == 1)
    def _():
        @pl.when(f_idx == 0)
        def _():
            out_ref[...] = jnp.zeros_like(out_ref)

        xb = x_ref[...]
        h1 = lax.dot_general(xb, w1_ref[0], (((1,), (1,)), ((), ())),
                             preferred_element_type=jnp.float32)
        h3 = lax.dot_general(xb, w3_ref[0], (((1,), (1,)), ((), ())),
                             preferred_element_type=jnp.float32)
        hg = h1 * lax.logistic(h1) * h3
        out_ref[...] += lax.dot_general(hg, w2_ref[0],
                                        (((1,), (0,)), ((), ())),
                                        preferred_element_type=jnp.float32)


def _sc_unsort_body(os_hbm, pe_hbm, po_hbm, outa_hbm, outb_hbm,
                    idx2_v, gbuf_v):
    wid = lax.axis_index("s") * NC + lax.axis_index("c")
    base = wid * TPW
    for p_hbm, o_hbm in ((pe_hbm, outa_hbm), (po_hbm, outb_hbm)):
        pltpu.sync_copy(p_hbm.at[pl.ds(base, TPW)], idx2_v)
        for ch in range(NCH):
            t0 = base + ch * CH
            pltpu.sync_copy(os_hbm.at[idx2_v.at[pl.ds(ch * CH, CH)]], gbuf_v)
            pltpu.sync_copy(gbuf_v, o_hbm.at[pl.ds(t0, CH)])


def _combine_body(nr_ref, a_ref, b_ref, w_ref, out_ref):
    out_ref[...] = nr_ref[...] + w_ref[...] * (a_ref[...] + b_ref[...])


_sc_dispatch = pl.kernel(
    _sc_dispatch_body,
    out_type=jax.ShapeDtypeStruct((NPAD, DIM), jnp.float32),
    mesh=_MESH,
    scratch_types=[pltpu.VMEM((TPW,), jnp.int32),
                   pltpu.VMEM((TPW,), jnp.int32),
                   pltpu.VMEM((CH, DIM), jnp.float32)])

_sc_unsort = pl.kernel(
    _sc_unsort_body,
    out_type=[jax.ShapeDtypeStruct((T, DIM), jnp.float32),
              jax.ShapeDtypeStruct((T, DIM), jnp.float32)],
    mesh=_MESH,
    scratch_types=[pltpu.VMEM((TPW,), jnp.int32),
                   pltpu.VMEM((CH, DIM), jnp.float32)])


@jax.jit
def kernel(x, next_r, gate_w, w1, w2, w3):
    tw, pe2, po2, te2, act2 = pl.pallas_call(
        _route_rank_body,
        grid=(T // TILE_A,),
        in_specs=[
            pl.BlockSpec((TILE_A, DIM), lambda g: (g, 0)),
            pl.BlockSpec((E, DIM), lambda g: (0, 0)),
        ],
        out_specs=[
            pl.BlockSpec((TILE_A, K), lambda g: (g, 0)),
            pl.BlockSpec((T, 1), lambda g: (0, 0)),
            pl.BlockSpec((T, 1), lambda g: (0, 0)),
            pl.BlockSpec((TE_LEN, 1), lambda g: (0, 0)),
            pl.BlockSpec((TE_LEN, 1), lambda g: (0, 0)),
        ],
        out_shape=[
            jax.ShapeDtypeStruct((T, K), jnp.float32),
            jax.ShapeDtypeStruct((T, 1), jnp.int32),
            jax.ShapeDtypeStruct((T, 1), jnp.int32),
            jax.ShapeDtypeStruct((TE_LEN, 1), jnp.int32),
            jax.ShapeDtypeStruct((TE_LEN, 1), jnp.int32),
        ],
        scratch_shapes=[
            pltpu.VMEM((2 * T, E), jnp.float32),
        ],
    )(x, gate_w)

    w_used = tw[: T // K].reshape(T, 1)

    pe = pe2.reshape(T)
    po = po2.reshape(T)
    te = te2.reshape(TE_LEN)
    act = act2.reshape(TE_LEN)

    xs = _sc_dispatch(x, pe, po)

    outs = pl.pallas_call(
        _ffn_body,
        grid_spec=pltpu.PrefetchScalarGridSpec(
            num_scalar_prefetch=2,
            grid=(NTILES, NDFF),
            in_specs=[
                pl.BlockSpec((TILE_F, DIM),
                             lambda g, f, te_r, a_r: (g * a_r[g], 0)),
                pl.BlockSpec((1, DFFB, DIM),
                             lambda g, f, te_r, a_r: (te_r[g], f, 0)),
                pl.BlockSpec((1, DFFB, DIM),
                             lambda g, f, te_r, a_r: (te_r[g], f, 0)),
                pl.BlockSpec((1, DFFB, DIM),
                             lambda g, f, te_r, a_r: (te_r[g], f, 0)),
            ],
            out_specs=pl.BlockSpec((TILE_F, DIM),
                                   lambda g, f, te_r, a_r: (g, 0)),
        ),
        out_shape=jax.ShapeDtypeStruct((NPAD, DIM), jnp.float32),
        compiler_params=pltpu.CompilerParams(
            vmem_limit_bytes=63 * 1024 * 1024),
    )(te, act, xs, w1, w3, w2)

    outa, outb = _sc_unsort(outs, pe, po)

    out = pl.pallas_call(
        _combine_body,
        grid=(4,),
        in_specs=[
            pl.BlockSpec((T // 4, DIM), lambda g: (g, 0)),
            pl.BlockSpec((T // 4, DIM), lambda g: (g, 0)),
            pl.BlockSpec((T // 4, DIM), lambda g: (g, 0)),
            pl.BlockSpec((T // 4, 1), lambda g: (g, 0)),
        ],
        out_specs=pl.BlockSpec((T // 4, DIM), lambda g: (g, 0)),
        out_shape=jax.ShapeDtypeStruct((T, DIM), jnp.float32),
    )(next_r, outa, outb, w_used)
    return out


# speedup vs baseline: 1.1798x; 1.0122x over previous
import jax
import jax.numpy as jnp
from jax import lax
from jax.experimental import pallas as pl
from jax.experimental.pallas import tpu as pltpu
from jax.experimental.pallas import tpu_sc as plsc

T = 4096
DIM = 1024
DFF = 2048
E = 8
K = 2

TILE_A = 1024
TILE_F = 256
NTILES = (T * K) // TILE_F + E
NPAD = NTILES * TILE_F
DFFB = 2048
NDFF = DFF // DFFB
TE_LEN = 80

RCH = 1024
NRCH = (T * K) // RCH

NC = 2
NS = 16
NW = NC * NS
TPW = T // NW
CH = 64
NCH = TPW // CH

_MESH = plsc.VectorSubcoreMesh(core_axis_name="c", subcore_axis_name="s",
                               num_cores=NC, num_subcores=NS)


def _route_rank_body(x_ref, gw_ref, tw_ref, pe_ref, po_ref, te_ref, act_ref,
                     ohi_s):
    g = pl.program_id(0)
    xb = x_ref[...]
    logits = lax.dot_general(xb, gw_ref[...], (((1,), (1,)), ((), ())),
                             preferred_element_type=jnp.float32)
    eids = lax.broadcasted_iota(jnp.int32, (1, E), 1)
    v1 = jnp.max(logits, axis=1, keepdims=True)
    i1 = jnp.argmax(logits, axis=1, keepdims=True).astype(jnp.int32)
    masked = jnp.where(eids == i1, -jnp.inf, logits)
    v2 = jnp.max(masked, axis=1, keepdims=True)
    i2 = jnp.argmax(masked, axis=1, keepdims=True).astype(jnp.int32)
    e2 = jnp.exp(v2 - v1)
    denom = 1.0 + e2
    tw_ref[...] = jnp.concatenate([1.0 / denom, e2 / denom], axis=1)

    oh0 = jnp.where(i1 == eids, 1.0, 0.0)
    oh1 = jnp.where(i2 == eids, 1.0, 0.0)
    ohg = jnp.concatenate(
        [oh0.reshape(TILE_A, 1, E), oh1.reshape(TILE_A, 1, E)], axis=1
    ).reshape(2 * TILE_A, E)
    ohi_s[pl.ds(g * 2 * TILE_A, 2 * TILE_A), :] = ohg

    @pl.when(g == (T // TILE_A) - 1)
    def _():
        ohi = ohi_s[...]
        tot = jnp.sum(ohi, axis=0, keepdims=True)
        padded = jnp.floor((tot + (TILE_F - 1)) * (1.0 / TILE_F)) * TILE_F
        r8 = lax.broadcasted_iota(jnp.int32, (E, E), 0)
        c8 = lax.broadcasted_iota(jnp.int32, (E, E), 1)
        u8 = jnp.where(r8 < c8, 1.0, 0.0)
        pad_base = lax.dot_general(padded, u8, (((1,), (0,)), ((), ())),
                                   preferred_element_type=jnp.float32)
        total_rows = jnp.sum(padded)

        rr = lax.broadcasted_iota(jnp.int32, (RCH, RCH), 0)
        cc = lax.broadcasted_iota(jnp.int32, (RCH, RCH), 1)
        lt = jnp.where(cc < rr, 1.0, 0.0)

        carry = jnp.zeros((1, E), jnp.float32)
        for c in range(NRCH):
            chunk = ohi[c * RCH:(c + 1) * RCH]
            within = lax.dot_general(lt, chunk, (((1,), (0,)), ((), ())),
                                     preferred_element_type=jnp.float32)
            base_vec = pad_base + carry
            basep = jnp.sum(chunk * base_vec, axis=1, keepdims=True)
            withinp = jnp.sum(within * chunk, axis=1, keepdims=True)
            pos = (basep + withinp).astype(jnp.int32).reshape(RCH // K, K)
            pe_ref[pl.ds(c * (RCH // K), RCH // K), :] = pos[:, 0:1]
            po_ref[pl.ds(c * (RCH // K), RCH // K), :] = pos[:, 1:2]
            carry = carry + jnp.sum(chunk, axis=0, keepdims=True)

        gi = lax.broadcasted_iota(jnp.int32, (TE_LEN, E), 0)
        ge = jnp.where((gi * TILE_F).astype(jnp.float32) >= pad_base,
                       1.0, 0.0)
        te_ref[...] = (jnp.sum(ge, axis=1, keepdims=True)
                       - 1.0).astype(jnp.int32)
        act_ref[...] = jnp.where(
            (gi[:, 0:1] * TILE_F).astype(jnp.float32) < total_rows,
            1, 0).astype(jnp.int32)


def _sc_dispatch_body(x_hbm, pe_hbm, po_hbm, xs_hbm, pe2_v, po2_v, xbuf_v):
    wid = lax.axis_index("s") * NC + lax.axis_index("c")
    base = wid * TPW
    pltpu.sync_copy(pe_hbm.at[pl.ds(base, TPW)], pe2_v)
    pltpu.sync_copy(po_hbm.at[pl.ds(base, TPW)], po2_v)
    for ch in range(NCH):
        pltpu.sync_copy(x_hbm.at[pl.ds(base + ch * CH, CH)], xbuf_v)
        pltpu.sync_copy(xbuf_v, xs_hbm.at[pe2_v.at[pl.ds(ch * CH, CH)]])
        pltpu.sync_copy(xbuf_v, xs_hbm.at[po2_v.at[pl.ds(ch * CH, CH)]])


def _ffn_body(te_ref, act_ref, x_ref, w1_ref, w3_ref, w2_ref, out_ref):
    g_idx = pl.program_id(0)
    f_idx = pl.program_id(1)

    @pl.when(act_ref[g_idx] == 1)
    def _():
        @pl.when(f_idx == 0)
        def _():
            out_ref[...] = jnp.zeros_like(out_ref)

        xb = x_ref[...]
        h1 = lax.dot_general(xb, w1_ref[0], (((1,), (1,)), ((), ())),
                             preferred_element_type=jnp.float32)
        h3 = lax.dot_general(xb, w3_ref[0], (((1,), (1,)), ((), ())),
                             preferred_element_type=jnp.float32)
        hg = h1 * lax.logistic(h1) * h3
        out_ref[...] += lax.dot_general(hg, w2_ref[0],
                                        (((1,), (0,)), ((), ())),
                                        preferred_element_type=jnp.float32)


def _sc_unsort_body(os_hbm, pe_hbm, po_hbm, outa_hbm, outb_hbm,
                    idx2_v, gbuf_v):
    wid = lax.axis_index("s") * NC + lax.axis_index("c")
    base = wid * TPW
    for p_hbm, o_hbm in ((pe_hbm, outa_hbm), (po_hbm, outb_hbm)):
        pltpu.sync_copy(p_hbm.at[pl.ds(base, TPW)], idx2_v)
        for ch in range(NCH):
            t0 = base + ch * CH
            pltpu.sync_copy(os_hbm.at[idx2_v.at[pl.ds(ch * CH, CH)]], gbuf_v)
            pltpu.sync_copy(gbuf_v, o_hbm.at[pl.ds(t0, CH)])


def _combine_body(nr_ref, a_ref, b_ref, w_ref, out_ref):
    out_ref[...] = nr_ref[...] + w_ref[...] * (a_ref[...] + b_ref[...])


_sc_dispatch = pl.kernel(
    _sc_dispatch_body,
    out_type=jax.ShapeDtypeStruct((NPAD, DIM), jnp.float32),
    mesh=_MESH,
    scratch_types=[pltpu.VMEM((TPW,), jnp.int32),
                   pltpu.VMEM((TPW,), jnp.int32),
                   pltpu.VMEM((CH, DIM), jnp.float32)])

_sc_unsort = pl.kernel(
    _sc_unsort_body,
    out_type=[jax.ShapeDtypeStruct((T, DIM), jnp.float32),
              jax.ShapeDtypeStruct((T, DIM), jnp.float32)],
    mesh=_MESH,
    scratch_types=[pltpu.VMEM((TPW,), jnp.int32),
                   pltpu.VMEM((CH, DIM), jnp.float32)])


@jax.jit
def kernel(x, next_r, gate_w, w1, w2, w3):
    tw, pe2, po2, te2, act2 = pl.pallas_call(
        _route_rank_body,
        grid=(T // TILE_A,),
        in_specs=[
            pl.BlockSpec((TILE_A, DIM), lambda g: (g, 0)),
            pl.BlockSpec((E, DIM), lambda g: (0, 0)),
        ],
        out_specs=[
            pl.BlockSpec((TILE_A, K), lambda g: (g, 0)),
            pl.BlockSpec((T, 1), lambda g: (0, 0)),
            pl.BlockSpec((T, 1), lambda g: (0, 0)),
            pl.BlockSpec((TE_LEN, 1), lambda g: (0, 0)),
            pl.BlockSpec((TE_LEN, 1), lambda g: (0, 0)),
        ],
        out_shape=[
            jax.ShapeDtypeStruct((T, K), jnp.float32),
            jax.ShapeDtypeStruct((T, 1), jnp.int32),
            jax.ShapeDtypeStruct((T, 1), jnp.int32),
            jax.ShapeDtypeStruct((TE_LEN, 1), jnp.int32),
            jax.ShapeDtypeStruct((TE_LEN, 1), jnp.int32),
        ],
        scratch_shapes=[
            pltpu.VMEM((2 * T, E), jnp.float32),
        ],
    )(x, gate_w)

    w_used = tw[: T // K].reshape(T, 1)

    pe = pe2.reshape(T)
    po = po2.reshape(T)
    te = te2.reshape(TE_LEN)
    act = act2.reshape(TE_LEN)

    xs = _sc_dispatch(x, pe, po)

    outs = pl.pallas_call(
        _ffn_body,
        grid_spec=pltpu.PrefetchScalarGridSpec(
            num_scalar_prefetch=2,
            grid=(NTILES, NDFF),
            in_specs=[
                pl.BlockSpec((TILE_F, DIM),
                             lambda g, f, te_r, a_r: (g * a_r[g], 0)),
                pl.BlockSpec((1, DFFB, DIM),
                             lambda g, f, te_r, a_r: (te_r[g], f, 0)),
                pl.BlockSpec((1, DFFB, DIM),
                             lambda g, f, te_r, a_r: (te_r[g], f, 0)),
                pl.BlockSpec((1, DFFB, DIM),
                             lambda g, f, te_r, a_r: (te_r[g], f, 0)),
            ],
            out_specs=pl.BlockSpec((TILE_F, DIM),
                                   lambda g, f, te_r, a_r: (g, 0)),
        ),
        out_shape=jax.ShapeDtypeStruct((NPAD, DIM), jnp.float32),
        compiler_params=pltpu.CompilerParams(
            vmem_limit_bytes=63 * 1024 * 1024),
    )(te, act, xs, w1, w3, w2)

    outa, outb = _sc_unsort(outs, pe, po)

    out = pl.pallas_call(
        _combine_body,
        grid=(4,),
        in_specs=[
            pl.BlockSpec((T // 4, DIM), lambda g: (g, 0)),
            pl.BlockSpec((T // 4, DIM), lambda g: (g, 0)),
            pl.BlockSpec((T // 4, DIM), lambda g: (g, 0)),
            pl.BlockSpec((T // 4, 1), lambda g: (g, 0)),
        ],
        out_specs=pl.BlockSpec((T // 4, DIM), lambda g: (g, 0)),
        out_shape=jax.ShapeDtypeStruct((T, DIM), jnp.float32),
    )(next_r, outa, outb, w_used)
    return out
